# baseline (speedup 1.0000x reference)
import functools

import jax
import jax.numpy as jnp
from jax import lax
from jax.experimental import pallas as pl
from jax.experimental.pallas import tpu as pltpu
from jax.experimental.pallas import tpu_sc as plsc

D = 64
CHUNK = 128
K = 5
S = K * CHUNK


@functools.lru_cache(maxsize=None)
def _make_gather(B: int):
    info = plsc.get_sparse_core_info()
    NC, NS = info.num_cores, info.num_subcores
    NW = NC * NS
    b_per_w = B // NW
    n_steps = b_per_w // S
    n_outer = n_steps // 2
    idx_rows = b_per_w // CHUNK

    mesh = plsc.VectorSubcoreMesh(core_axis_name="c", subcore_axis_name="s")

    @functools.partial(
        pl.kernel,
        mesh=mesh,
        out_type=jax.ShapeDtypeStruct((B, D), jnp.float32),
        scratch_types=[
            pltpu.VMEM((idx_rows, CHUNK), jnp.int32),
            pltpu.VMEM((S, D), jnp.float32),
            pltpu.VMEM((S, D), jnp.float32),
            pltpu.SemaphoreType.DMA,
            pltpu.SemaphoreType.DMA,
            pltpu.SemaphoreType.DMA,
            pltpu.SemaphoreType.DMA,
        ],
        compiler_params=pltpu.CompilerParams(use_tc_tiling_on_sc=False),
    )
    def k(idx_hbm, table_hbm, out_hbm, idx_v, rows0, rows1, g0, g1, s0, s1):
        wid = lax.axis_index("s") * NC + lax.axis_index("c")
        base = wid * b_per_w
        rows = (rows0, rows1)
        gsem = (g0, g1)
        ssem = (s0, s1)

        pltpu.sync_copy(idx_hbm.at[pl.ds(wid * idx_rows, idx_rows)], idx_v)

        def fire_gathers(b, step):
            for t in range(K):
                pltpu.async_copy(
                    table_hbm.at[idx_v.at[step * K + t]],
                    rows[b].at[pl.ds(t * CHUNK, CHUNK)],
                    gsem[b],
                )

        def drain_gathers(b, step):
            for t in range(K):
                pltpu.make_async_copy(
                    table_hbm.at[idx_v.at[step * K + t]],
                    rows[b].at[pl.ds(t * CHUNK, CHUNK)],
                    gsem[b],
                ).wait()

        def fire_store(b, step):
            pltpu.make_async_copy(
                rows[b], out_hbm.at[pl.ds(base + step * S, S)], ssem[b]
            ).start()

        def drain_store(b, step):
            pltpu.make_async_copy(
                rows[b], out_hbm.at[pl.ds(base + step * S, S)], ssem[b]
            ).wait()

        fire_gathers(0, 0)

        def outer(o, carry):
            t0 = o * 2
            t1 = t0 + 1

            @pl.when(o > 0)
            def _():
                drain_store(1, t1 - 2)

            fire_gathers(1, t1)
            drain_gathers(0, t0)
            fire_store(0, t0)
            drain_store(0, t0)

            @pl.when(o + 1 < n_outer)
            def _():
                fire_gathers(0, t0 + 2)

            drain_gathers(1, t1)
            fire_store(1, t1)
            return carry

        lax.fori_loop(0, n_outer, outer, 0)
        drain_store(1, n_steps - 1)

    return k


def kernel(x, table):
    n, s = x.shape
    B = n * s
    idx2d = x.reshape(B // CHUNK, CHUNK).astype(jnp.int32)
    out = _make_gather(B)(idx2d, table.astype(jnp.float32))
    return out.reshape(n, s, D)


# speedup vs baseline: 1.3903x; 1.3903x over previous
import functools

import jax
import jax.numpy as jnp
from jax import lax
from jax.experimental import pallas as pl
from jax.experimental.pallas import tpu as pltpu
from jax.experimental.pallas import tpu_sc as plsc

D = 64
CHUNK = 128
K = 5
S = K * CHUNK


@functools.lru_cache(maxsize=None)
def _make_gather(B: int, V: int):
    info = plsc.get_sparse_core_info()
    NC, NS = info.num_cores, info.num_subcores
    NW = NC * NS
    b_per_w = B // NW
    n_steps = b_per_w // S
    n_outer = n_steps // 2
    idx_rows = b_per_w // CHUNK

    mesh = plsc.VectorSubcoreMesh(core_axis_name="c", subcore_axis_name="s")

    @functools.partial(
        pl.kernel,
        mesh=mesh,
        out_type=jax.ShapeDtypeStruct((B, D), jnp.float32),
        scratch_types=[
            pltpu.VMEM_SHARED((V, D), jnp.float32),
            pltpu.VMEM((idx_rows, CHUNK), jnp.int32),
            pltpu.VMEM((S, D), jnp.float32),
            pltpu.VMEM((S, D), jnp.float32),
            pltpu.SemaphoreType.DMA,
            pltpu.SemaphoreType.DMA,
            pltpu.SemaphoreType.DMA,
            pltpu.SemaphoreType.DMA,
        ],
        compiler_params=pltpu.CompilerParams(use_tc_tiling_on_sc=False),
    )
    def k(idx_hbm, table_hbm, out_hbm, table_sp, idx_v, rows0, rows1,
          g0, g1, s0, s1):
        sid = lax.axis_index("s")
        wid = sid * NC + lax.axis_index("c")
        base = wid * b_per_w
        rows = (rows0, rows1)
        gsem = (g0, g1)
        ssem = (s0, s1)

        @pl.when(sid == 0)
        def _():
            pltpu.sync_copy(table_hbm, table_sp)

        pltpu.sync_copy(idx_hbm.at[pl.ds(wid * idx_rows, idx_rows)], idx_v)
        plsc.subcore_barrier()

        def fire_gathers(b, step):
            for t in range(K):
                pltpu.async_copy(
                    table_sp.at[idx_v.at[step * K + t]],
                    rows[b].at[pl.ds(t * CHUNK, CHUNK)],
                    gsem[b],
                )

        def drain_gathers(b, step):
            for t in range(K):
                pltpu.make_async_copy(
                    table_sp.at[idx_v.at[step * K + t]],
                    rows[b].at[pl.ds(t * CHUNK, CHUNK)],
                    gsem[b],
                ).wait()

        def fire_store(b, step):
            pltpu.make_async_copy(
                rows[b], out_hbm.at[pl.ds(base + step * S, S)], ssem[b]
            ).start()

        def drain_store(b, step):
            pltpu.make_async_copy(
                rows[b], out_hbm.at[pl.ds(base + step * S, S)], ssem[b]
            ).wait()

        fire_gathers(0, 0)

        def outer(o, carry):
            t0 = o * 2
            t1 = t0 + 1

            @pl.when(o > 0)
            def _():
                drain_store(1, t1 - 2)

            fire_gathers(1, t1)
            drain_gathers(0, t0)
            fire_store(0, t0)
            drain_store(0, t0)

            @pl.when(o + 1 < n_outer)
            def _():
                fire_gathers(0, t0 + 2)

            drain_gathers(1, t1)
            fire_store(1, t1)
            return carry

        lax.fori_loop(0, n_outer, outer, 0)
        drain_store(1, n_steps - 1)

    return k


def kernel(x, table):
    n, s = x.shape
    B = n * s
    V = table.shape[0]
    idx2d = x.reshape(B // CHUNK, CHUNK).astype(jnp.int32)
    out = _make_gather(B, V)(idx2d, table.astype(jnp.float32))
    return out.reshape(n, s, D)
